# baseline (speedup 1.0000x reference)
import functools

import jax
import jax.numpy as jnp
from jax import lax
from jax.experimental import pallas as pl
from jax.experimental.pallas import tpu as pltpu
from jax.experimental.pallas import tpu_sc as plsc

N = 10000
E = 320000
D = 128

NC = 2
NS = 16
NW = NC * NS
C = 64
G = 160
W = 16
NWIN = G // W
NB = 4
EPW = G * C
EPAD = NW * EPW
NP = 10240
RPT = NP // NS
BR = 512
GRID = NP // BR

_mesh = plsc.VectorSubcoreMesh(core_axis_name="c", subcore_axis_name="s")


DW = D
CD = 128
GD = EPW // CD
WD = 16
NWIND = GD // WD


@functools.partial(
    pl.kernel,
    mesh=_mesh,
    out_type=jax.ShapeDtypeStruct((NC, NP, DW), jnp.float32),
    scratch_types=[
        pltpu.VMEM((NWIND, WD, CD), jnp.int32),
        pltpu.VMEM((CD, DW), jnp.float32),
        pltpu.VMEM_SHARED((NP, DW), jnp.float32),
        pltpu.SemaphoreType.DMA,
    ],
)
def _deg_kernel(dst_hbm, ones_hbm, z_hbm, degp_hbm, dstv, onesv, degs, ssem):
    c = lax.axis_index("c")
    s = lax.axis_index("s")
    wid = c * NS + s
    pltpu.sync_copy(dst_hbm.at[wid], dstv)
    pltpu.sync_copy(ones_hbm, onesv)
    rows = pl.ds(s * RPT, RPT)
    pltpu.sync_copy(z_hbm.at[rows], degs.at[rows])
    plsc.subcore_barrier()

    for w in range(NWIND):
        def fire(lg, carry):
            pltpu.async_copy(onesv, degs.at[dstv.at[w, lg]], ssem, add=True)
            return carry

        lax.fori_loop(0, WD, fire, 0)

        def drain(lg, carry):
            pltpu.make_async_copy(onesv, degs.at[dstv.at[0, 0]], ssem).wait()
            return carry

        lax.fori_loop(0, WD, drain, 0)
    plsc.subcore_barrier()
    pltpu.sync_copy(degs.at[rows], degp_hbm.at[c, rows])


@functools.partial(
    pl.kernel,
    mesh=_mesh,
    out_type=jax.ShapeDtypeStruct((NC, NP, D), jnp.float32),
    scratch_types=[
        pltpu.VMEM((2, W, C), jnp.int32),
        pltpu.VMEM((2, W, C), jnp.int32),
        pltpu.VMEM((NB, C, D), jnp.float32),
        pltpu.VMEM_SHARED((NP, D), jnp.float32),
        pltpu.SemaphoreType.DMA,
        pltpu.SemaphoreType.DMA,
        pltpu.SemaphoreType.DMA,
        pltpu.SemaphoreType.DMA,
        pltpu.SemaphoreType.DMA,
        pltpu.SemaphoreType.DMA,
        pltpu.SemaphoreType.DMA,
        pltpu.SemaphoreType.DMA,
        pltpu.SemaphoreType.DMA,
        pltpu.SemaphoreType.DMA,
    ],
)
def _edge_kernel(y_hbm, z_hbm, src_hbm, dst_hbm, p_hbm, srcw, dstw, bufs, acc,
                 g0, g1, g2, g3, s0, s1, s2, s3, wsem0, wsem1):
    c = lax.axis_index("c")
    s = lax.axis_index("s")
    wid = c * NS + s
    gsems = (g0, g1, g2, g3)
    ssems = (s0, s1, s2, s3)
    wsems = (wsem0, wsem1)

    def load_window(wi, sl):
        pltpu.async_copy(src_hbm.at[wid, wi], srcw.at[sl], wsems[sl])
        pltpu.async_copy(dst_hbm.at[wid, wi], dstw.at[sl], wsems[sl])

    def wait_window(wi, sl):
        pltpu.make_async_copy(src_hbm.at[wid, wi], srcw.at[sl],
                              wsems[sl]).wait()
        pltpu.make_async_copy(dst_hbm.at[wid, wi], dstw.at[sl],
                              wsems[sl]).wait()

    def gather(sl, lg, b):
        pltpu.async_copy(y_hbm.at[srcw.at[sl, lg]], bufs.at[b], gsems[b])

    def wait_gather(sl, lg, b):
        pltpu.make_async_copy(y_hbm.at[srcw.at[sl, lg]], bufs.at[b],
                              gsems[b]).wait()

    def scatter(sl, lg, b):
        pltpu.async_copy(bufs.at[b], acc.at[dstw.at[sl, lg]], ssems[b],
                         add=True)

    def wait_scatter(b):
        pltpu.make_async_copy(bufs.at[b], acc.at[dstw.at[0, 0]],
                              ssems[b]).wait()

    load_window(0, 0)

    rows = pl.ds(s * RPT, RPT)

    @pl.when(c == 0)
    def _():
        pltpu.sync_copy(y_hbm.at[rows], acc.at[rows])

    @pl.when(c == 1)
    def _():
        pltpu.sync_copy(z_hbm.at[rows], acc.at[rows])

    plsc.subcore_barrier()

    wait_window(0, 0)
    gather(0, 0, 0)
    gather(0, 1, 1)
    gather(0, 2, 2)

    def refill(pws, plg, b3):
        wait_scatter(b3)
        gather(pws, plg, b3)

    def window(w, ws, ww, is_even):
        wait_gather(ws, 0, 0)
        scatter(ws, 0, 0)
        if is_even:
            @pl.when(w > 0)
            def _():
                wait_scatter(3)
        else:
            wait_scatter(3)
        gather(ws, 3, 3)
        for j in (1, 2, 3):
            wait_gather(ws, j, j)
            scatter(ws, j, j)
            refill(ws, j + 3, (j + 3) % NB)
        if is_even:
            load_window(w + 1, 1 - ws)
        else:
            @pl.when(ww < NWIN // 2 - 1)
            def _():
                load_window(w + 1, 1 - ws)

        def mid(qq, carry):
            base = 4 + qq * 4
            for j in range(4):
                lg = base + j
                wait_gather(ws, lg, j)
                scatter(ws, lg, j)
                refill(ws, lg + 3, (j + 3) % NB)
            return carry

        lax.fori_loop(0, (W - 8) // 4, mid, 0)

        wait_gather(ws, W - 4, 0)
        scatter(ws, W - 4, 0)
        refill(ws, W - 1, 3)
        if is_even:
            wait_window(w + 1, 1 - ws)
        else:
            @pl.when(ww < NWIN // 2 - 1)
            def _():
                wait_window(w + 1, 1 - ws)
        for j in (1, 2, 3):
            lg = W - 4 + j
            wait_gather(ws, lg, j)
            scatter(ws, lg, j)
            b3 = (j + 3) % NB
            if is_even:
                wait_scatter(b3)
                gather(1 - ws, j - 1, b3)
            else:
                @pl.when(ww < NWIN // 2 - 1)
                def _():
                    wait_scatter(b3)
                    gather(1 - ws, j - 1, b3)

    def wpair(ww, carry):
        window(2 * ww, 0, ww, True)
        window(2 * ww + 1, 1, ww, False)
        return carry

    lax.fori_loop(0, NWIN // 2, wpair, 0)

    for b in range(NB):
        wait_scatter(b)

    plsc.subcore_barrier()
    pltpu.sync_copy(acc.at[rows], p_hbm.at[c, rows])


def _dinv_of(degp_block):
    deg = degp_block[0, :, 0] + degp_block[1, :, 0] + 1.0
    return lax.rsqrt(deg)


def _row_mask(i):
    r = i * BR + lax.broadcasted_iota(jnp.int32, (BR, 1), 0)
    return r < N


def _mm1_body(x_ref, degp_ref, w_ref, o_ref):
    i = pl.program_id(0)
    dinv = _dinv_of(degp_ref[...])
    xs = x_ref[...] * dinv[:, None]
    y = jnp.dot(xs, w_ref[...], preferred_element_type=jnp.float32)
    o_ref[...] = jnp.where(_row_mask(i), y, 0.0)


def _mm_mid_body(p_ref, degp_ref, b_ref, w_ref, o_ref):
    i = pl.program_id(0)
    dinv = _dinv_of(degp_ref[...])
    acc = p_ref[0] + p_ref[1]
    h = jnp.maximum(acc * dinv[:, None] + b_ref[...], 0.0)
    y = jnp.dot(h * dinv[:, None], w_ref[...],
                preferred_element_type=jnp.float32)
    o_ref[...] = jnp.where(_row_mask(i), y, 0.0)


def _fin_body(p_ref, degp_ref, b_ref, o_ref):
    dinv = _dinv_of(degp_ref[...])
    acc = p_ref[0] + p_ref[1]
    o_ref[...] = acc * dinv[:, None] + b_ref[...]


_degp_spec = pl.BlockSpec((NC, BR, DW), lambda i: (0, i, 0))
_p_spec = pl.BlockSpec((NC, BR, D), lambda i: (0, i, 0))
_row_spec = pl.BlockSpec((BR, D), lambda i: (i, 0))
_w_spec = pl.BlockSpec((D, D), lambda i: (0, 0))
_b_spec = pl.BlockSpec((1, D), lambda i: (0, 0))

_mm1 = pl.pallas_call(
    _mm1_body,
    grid=(GRID,),
    in_specs=[_row_spec, _degp_spec, _w_spec],
    out_specs=_row_spec,
    out_shape=jax.ShapeDtypeStruct((NP, D), jnp.float32),
)

_mm_mid = pl.pallas_call(
    _mm_mid_body,
    grid=(GRID,),
    in_specs=[_p_spec, _degp_spec, _b_spec, _w_spec],
    out_specs=_row_spec,
    out_shape=jax.ShapeDtypeStruct((NP, D), jnp.float32),
)

_fin = pl.pallas_call(
    _fin_body,
    grid=(GRID,),
    in_specs=[_p_spec, _degp_spec, _b_spec],
    out_specs=_row_spec,
    out_shape=jax.ShapeDtypeStruct((N, D), jnp.float32),
)


def kernel(x, edge_index, W1, b1, W2, b2, W3, b3):
    src = edge_index[0]
    dst = edge_index[1]
    pad = EPAD - E
    padv = N + jnp.arange(pad, dtype=jnp.int32) % (NP - N)
    srcp = jnp.concatenate([src.astype(jnp.int32), padv]).reshape(NW, G, C)
    dstp = jnp.concatenate([dst.astype(jnp.int32), padv]).reshape(NW, G, C)
    zeros = jnp.zeros((NP, D), jnp.float32)
    ones = jnp.ones((CD, DW), jnp.float32)
    b1r = b1.reshape(1, D)
    b2r = b2.reshape(1, D)
    b3r = b3.reshape(1, D)

    srcw = srcp.reshape(NW, NWIN, W, C)
    dstw = dstp.reshape(NW, NWIN, W, C)

    degp = _deg_kernel(dstp.reshape(NW, NWIND, WD, CD), ones, zeros)
    y1 = _mm1(x, degp, W1)
    p1 = _edge_kernel(y1, zeros, srcw, dstw)
    y2 = _mm_mid(p1, degp, b1r, W2)
    p2 = _edge_kernel(y2, zeros, srcw, dstw)
    y3 = _mm_mid(p2, degp, b2r, W3)
    p3 = _edge_kernel(y3, zeros, srcw, dstw)
    return _fin(p3, degp, b3r)


# speedup vs baseline: 1.0025x; 1.0025x over previous
import functools

import jax
import jax.numpy as jnp
from jax import lax
from jax.experimental import pallas as pl
from jax.experimental.pallas import tpu as pltpu
from jax.experimental.pallas import tpu_sc as plsc

N = 10000
E = 320000
D = 128

NC = 2
NS = 16
NW = NC * NS
C = 64
G = 160
W = 20
NWIN = G // W
NB = 4
EPW = G * C
EPAD = NW * EPW
NP = 10240
RPT = NP // NS
BR = 512
GRID = NP // BR

_mesh = plsc.VectorSubcoreMesh(core_axis_name="c", subcore_axis_name="s")


DW = D
CD = 128
GD = EPW // CD
WD = 16
NWIND = GD // WD


@functools.partial(
    pl.kernel,
    mesh=_mesh,
    out_type=jax.ShapeDtypeStruct((NC, NP, DW), jnp.float32),
    scratch_types=[
        pltpu.VMEM((NWIND, WD, CD), jnp.int32),
        pltpu.VMEM((CD, DW), jnp.float32),
        pltpu.VMEM_SHARED((NP, DW), jnp.float32),
        pltpu.SemaphoreType.DMA,
    ],
)
def _deg_kernel(dst_hbm, ones_hbm, z_hbm, degp_hbm, dstv, onesv, degs, ssem):
    c = lax.axis_index("c")
    s = lax.axis_index("s")
    wid = c * NS + s
    pltpu.sync_copy(dst_hbm.at[wid], dstv)
    pltpu.sync_copy(ones_hbm, onesv)
    rows = pl.ds(s * RPT, RPT)
    pltpu.sync_copy(z_hbm.at[rows], degs.at[rows])
    plsc.subcore_barrier()

    for w in range(NWIND):
        def fire(lg, carry):
            pltpu.async_copy(onesv, degs.at[dstv.at[w, lg]], ssem, add=True)
            return carry

        lax.fori_loop(0, WD, fire, 0)

        def drain(lg, carry):
            pltpu.make_async_copy(onesv, degs.at[dstv.at[0, 0]], ssem).wait()
            return carry

        lax.fori_loop(0, WD, drain, 0)
    plsc.subcore_barrier()
    pltpu.sync_copy(degs.at[rows], degp_hbm.at[c, rows])


@functools.partial(
    pl.kernel,
    mesh=_mesh,
    out_type=jax.ShapeDtypeStruct((NC, NP, D), jnp.float32),
    scratch_types=[
        pltpu.VMEM((2, W, C), jnp.int32),
        pltpu.VMEM((2, W, C), jnp.int32),
        pltpu.VMEM((NB, C, D), jnp.float32),
        pltpu.VMEM_SHARED((NP, D), jnp.float32),
        pltpu.SemaphoreType.DMA,
        pltpu.SemaphoreType.DMA,
        pltpu.SemaphoreType.DMA,
        pltpu.SemaphoreType.DMA,
        pltpu.SemaphoreType.DMA,
        pltpu.SemaphoreType.DMA,
        pltpu.SemaphoreType.DMA,
        pltpu.SemaphoreType.DMA,
        pltpu.SemaphoreType.DMA,
        pltpu.SemaphoreType.DMA,
    ],
)
def _edge_kernel(y_hbm, z_hbm, src_hbm, dst_hbm, p_hbm, srcw, dstw, bufs, acc,
                 g0, g1, g2, g3, s0, s1, s2, s3, wsem0, wsem1):
    c = lax.axis_index("c")
    s = lax.axis_index("s")
    wid = c * NS + s
    gsems = (g0, g1, g2, g3)
    ssems = (s0, s1, s2, s3)
    wsems = (wsem0, wsem1)

    def load_window(wi, sl):
        pltpu.async_copy(src_hbm.at[wid, wi], srcw.at[sl], wsems[sl])
        pltpu.async_copy(dst_hbm.at[wid, wi], dstw.at[sl], wsems[sl])

    def wait_window(wi, sl):
        pltpu.make_async_copy(src_hbm.at[wid, wi], srcw.at[sl],
                              wsems[sl]).wait()
        pltpu.make_async_copy(dst_hbm.at[wid, wi], dstw.at[sl],
                              wsems[sl]).wait()

    def gather(sl, lg, b):
        pltpu.async_copy(y_hbm.at[srcw.at[sl, lg]], bufs.at[b], gsems[b])

    def wait_gather(sl, lg, b):
        pltpu.make_async_copy(y_hbm.at[srcw.at[sl, lg]], bufs.at[b],
                              gsems[b]).wait()

    def scatter(sl, lg, b):
        pltpu.async_copy(bufs.at[b], acc.at[dstw.at[sl, lg]], ssems[b],
                         add=True)

    def wait_scatter(b):
        pltpu.make_async_copy(bufs.at[b], acc.at[dstw.at[0, 0]],
                              ssems[b]).wait()

    load_window(0, 0)

    rows = pl.ds(s * RPT, RPT)

    wait_window(0, 0)
    gather(0, 0, 0)
    gather(0, 1, 1)
    gather(0, 2, 2)

    @pl.when(c == 0)
    def _():
        pltpu.sync_copy(y_hbm.at[rows], acc.at[rows])

    @pl.when(c == 1)
    def _():
        pltpu.sync_copy(z_hbm.at[rows], acc.at[rows])

    plsc.subcore_barrier()

    def refill(pws, plg, b3):
        wait_scatter(b3)
        gather(pws, plg, b3)

    def window(w, ws, ww, is_even):
        wait_gather(ws, 0, 0)
        scatter(ws, 0, 0)
        if is_even:
            @pl.when(w > 0)
            def _():
                wait_scatter(3)
        else:
            wait_scatter(3)
        gather(ws, 3, 3)
        for j in (1, 2, 3):
            wait_gather(ws, j, j)
            scatter(ws, j, j)
            refill(ws, j + 3, (j + 3) % NB)
        if is_even:
            load_window(w + 1, 1 - ws)
        else:
            @pl.when(ww < NWIN // 2 - 1)
            def _():
                load_window(w + 1, 1 - ws)

        def mid(qq, carry):
            base = 4 + qq * 4
            for j in range(4):
                lg = base + j
                wait_gather(ws, lg, j)
                scatter(ws, lg, j)
                refill(ws, lg + 3, (j + 3) % NB)
            return carry

        lax.fori_loop(0, (W - 8) // 4, mid, 0)

        wait_gather(ws, W - 4, 0)
        scatter(ws, W - 4, 0)
        refill(ws, W - 1, 3)
        if is_even:
            wait_window(w + 1, 1 - ws)
        else:
            @pl.when(ww < NWIN // 2 - 1)
            def _():
                wait_window(w + 1, 1 - ws)
        for j in (1, 2, 3):
            lg = W - 4 + j
            wait_gather(ws, lg, j)
            scatter(ws, lg, j)
            b3 = (j + 3) % NB
            if is_even:
                wait_scatter(b3)
                gather(1 - ws, j - 1, b3)
            else:
                @pl.when(ww < NWIN // 2 - 1)
                def _():
                    wait_scatter(b3)
                    gather(1 - ws, j - 1, b3)

    def wpair(ww, carry):
        window(2 * ww, 0, ww, True)
        window(2 * ww + 1, 1, ww, False)
        return carry

    lax.fori_loop(0, NWIN // 2, wpair, 0)

    for b in range(NB):
        wait_scatter(b)

    plsc.subcore_barrier()
    pltpu.sync_copy(acc.at[rows], p_hbm.at[c, rows])


def _dinv_of(degp_block):
    deg = degp_block[0, :, 0] + degp_block[1, :, 0] + 1.0
    return lax.rsqrt(deg)


def _row_mask(i):
    r = i * BR + lax.broadcasted_iota(jnp.int32, (BR, 1), 0)
    return r < N


def _mm1_body(x_ref, degp_ref, w_ref, o_ref):
    i = pl.program_id(0)
    dinv = _dinv_of(degp_ref[...])
    xs = x_ref[...] * dinv[:, None]
    y = jnp.dot(xs, w_ref[...], preferred_element_type=jnp.float32)
    o_ref[...] = jnp.where(_row_mask(i), y, 0.0)


def _mm_mid_body(p_ref, degp_ref, b_ref, w_ref, o_ref):
    i = pl.program_id(0)
    dinv = _dinv_of(degp_ref[...])
    acc = p_ref[0] + p_ref[1]
    h = jnp.maximum(acc * dinv[:, None] + b_ref[...], 0.0)
    y = jnp.dot(h * dinv[:, None], w_ref[...],
                preferred_element_type=jnp.float32)
    o_ref[...] = jnp.where(_row_mask(i), y, 0.0)


def _fin_body(p_ref, degp_ref, b_ref, o_ref):
    dinv = _dinv_of(degp_ref[...])
    acc = p_ref[0] + p_ref[1]
    o_ref[...] = acc * dinv[:, None] + b_ref[...]


_degp_spec = pl.BlockSpec((NC, BR, DW), lambda i: (0, i, 0))
_p_spec = pl.BlockSpec((NC, BR, D), lambda i: (0, i, 0))
_row_spec = pl.BlockSpec((BR, D), lambda i: (i, 0))
_w_spec = pl.BlockSpec((D, D), lambda i: (0, 0))
_b_spec = pl.BlockSpec((1, D), lambda i: (0, 0))

_mm1 = pl.pallas_call(
    _mm1_body,
    grid=(GRID,),
    in_specs=[_row_spec, _degp_spec, _w_spec],
    out_specs=_row_spec,
    out_shape=jax.ShapeDtypeStruct((NP, D), jnp.float32),
)

_mm_mid = pl.pallas_call(
    _mm_mid_body,
    grid=(GRID,),
    in_specs=[_p_spec, _degp_spec, _b_spec, _w_spec],
    out_specs=_row_spec,
    out_shape=jax.ShapeDtypeStruct((NP, D), jnp.float32),
)

_fin = pl.pallas_call(
    _fin_body,
    grid=(GRID,),
    in_specs=[_p_spec, _degp_spec, _b_spec],
    out_specs=_row_spec,
    out_shape=jax.ShapeDtypeStruct((N, D), jnp.float32),
)


def kernel(x, edge_index, W1, b1, W2, b2, W3, b3):
    src = edge_index[0]
    dst = edge_index[1]
    pad = EPAD - E
    padv = N + jnp.arange(pad, dtype=jnp.int32) % (NP - N)
    srcp = jnp.concatenate([src.astype(jnp.int32), padv]).reshape(NW, G, C)
    dstp = jnp.concatenate([dst.astype(jnp.int32), padv]).reshape(NW, G, C)
    zeros = jnp.zeros((NP, D), jnp.float32)
    ones = jnp.ones((CD, DW), jnp.float32)
    b1r = b1.reshape(1, D)
    b2r = b2.reshape(1, D)
    b3r = b3.reshape(1, D)

    srcw = srcp.reshape(NW, NWIN, W, C)
    dstw = dstp.reshape(NW, NWIN, W, C)

    degp = _deg_kernel(dstp.reshape(NW, NWIND, WD, CD), ones, zeros)
    y1 = _mm1(x, degp, W1)
    p1 = _edge_kernel(y1, zeros, srcw, dstw)
    y2 = _mm_mid(p1, degp, b1r, W2)
    p2 = _edge_kernel(y2, zeros, srcw, dstw)
    y3 = _mm_mid(p2, degp, b2r, W3)
    p3 = _edge_kernel(y3, zeros, srcw, dstw)
    return _fin(p3, degp, b3r)
